# Initial kernel scaffold; baseline (speedup 1.0000x reference)
import jax
import jax.numpy as jnp
from jax.experimental import pallas as pl


def kernel(x, Wc, bc, Wp, bp):
    raise NotImplementedError("write your pallas kernel here")


# speedup vs baseline: 6.3225x; 6.3225x over previous
import jax
import jax.numpy as jnp
from jax import lax
from jax.experimental import pallas as pl

_K = 4
_S = 256
_C = 392
_RO = 4 * 96
_TN = 256


def _zf_body(xs_ref, wz_ref, zf_ref):
    zf_ref[0] = lax.dot_general(
        xs_ref[0], wz_ref[...], (((1,), (0,)), ((), ())),
        preferred_element_type=jnp.float32)


def _main_body(x1t_ref, xst_ref, zf_ref, b2_ref, out_ref):
    xt = x1t_ref[0]
    xst = xst_ref[0]
    sn2 = jnp.sum(xst * xst, axis=0, keepdims=True)
    inner = lax.dot_general(xt, xst, (((1,), (0,)), ((), ())),
                            preferred_element_type=jnp.float32)
    neg = 2.0 * inner - sn2
    iota_s = lax.broadcasted_iota(jnp.int32, (_TN, _S), 1)
    acc = jnp.full((_TN, _RO), 0.0, jnp.float32)
    for k in range(_K):
        m = jnp.max(neg, axis=1, keepdims=True)
        hit = neg == m
        idx = jnp.min(jnp.where(hit, iota_s, _S), axis=1, keepdims=True)
        sel = iota_s == idx
        oh = sel.astype(jnp.float32)
        acc = acc + lax.dot_general(
            oh, zf_ref[0][:, k * _RO:(k + 1) * _RO],
            (((1,), (0,)), ((), ())), preferred_element_type=jnp.float32)
        neg = jnp.where(sel, -jnp.inf, neg)
    out_ref[0] = acc + b2_ref[...]


def kernel(x, Wc, bc, Wp, bp):
    B, Cin, H, W = x.shape
    Hu, Wu = H // 2, W // 2
    N = Hu * Wu
    xg, yg = jnp.meshgrid(jnp.arange(H, dtype=jnp.float32),
                          jnp.arange(W, dtype=jnp.float32), indexing='ij')
    nrm = jnp.maximum(jnp.sqrt(xg * xg + yg * yg), 1e-12)
    coords = jnp.broadcast_to(jnp.stack([xg / nrm, yg / nrm])[None],
                              (B, 2, H, W))
    xcat = jnp.concatenate([x, coords], axis=1)
    x1t = (xcat.reshape(B, Cin + 2, Hu, 2, Wu, 2)
           .transpose(0, 2, 4, 1, 3, 5).reshape(B, N, _C))
    ind = jnp.round(jnp.linspace(0, Hu - 1, 16)).astype(jnp.int32)
    flat = (ind[:, None] * Wu + ind[None, :]).reshape(-1)
    xs = jnp.take(x1t, flat, axis=1)
    xst = xs.transpose(0, 2, 1)
    Wc4 = Wc.reshape((Cin + 2), 4, _C, _K)
    Wz = jnp.einsum('op,prck->ckro', Wp, Wc4).reshape(_C, _K * _RO)
    b2 = (jnp.einsum('op,pr->ro', Wp, bc.reshape(Cin + 2, 4))
          + bp[None, :]).reshape(1, _RO)

    zf = pl.pallas_call(
        _zf_body,
        grid=(B,),
        in_specs=[
            pl.BlockSpec((1, _S, _C), lambda b: (b, 0, 0)),
            pl.BlockSpec((_C, _K * _RO), lambda b: (0, 0)),
        ],
        out_specs=pl.BlockSpec((1, _S, _K * _RO), lambda b: (b, 0, 0)),
        out_shape=jax.ShapeDtypeStruct((B, _S, _K * _RO), jnp.float32),
    )(xs, Wz)

    out = pl.pallas_call(
        _main_body,
        grid=(B, N // _TN),
        in_specs=[
            pl.BlockSpec((1, _TN, _C), lambda b, i: (b, i, 0)),
            pl.BlockSpec((1, _C, _S), lambda b, i: (b, 0, 0)),
            pl.BlockSpec((1, _S, _K * _RO), lambda b, i: (b, 0, 0)),
            pl.BlockSpec((1, _RO), lambda b, i: (0, 0)),
        ],
        out_specs=pl.BlockSpec((1, _TN, _RO), lambda b, i: (b, i, 0)),
        out_shape=jax.ShapeDtypeStruct((B, N, _RO), jnp.float32),
    )(x1t, xst, zf, b2)

    return (out.reshape(B, Hu, Wu, 2, 2, 96)
            .transpose(0, 5, 1, 3, 2, 4).reshape(B, 96, H, W))


# speedup vs baseline: 15.9690x; 2.5257x over previous
import jax
import jax.numpy as jnp
from jax import lax
from jax.experimental import pallas as pl

_K = 4
_S = 256
_C = 392
_RO = 4 * 96
_R = 4
_TN = _R * 112


def _zft_body(wzt_ref, xst_ref, zft_ref):
    zft_ref[0] = lax.dot_general(
        wzt_ref[...], xst_ref[0], (((1,), (0,)), ((), ())),
        preferred_element_type=jnp.float32)


def _mm(a, b):
    return lax.dot_general(a, b, (((1,), (0,)), ((), ())),
                           preferred_element_type=jnp.float32)


def _fused_body(x_ref, xsij_ref, zft_ref, b2_ref, out_ref):
    f32 = jnp.float32
    i0 = pl.program_id(1)
    xb = x_ref[0]
    r224 = lax.broadcasted_iota(jnp.int32, (224, 112), 0)
    c224 = lax.broadcasted_iota(jnp.int32, (224, 112), 1)
    Pe = (r224 == 2 * c224).astype(f32)
    Po = (r224 == 2 * c224 + 1).astype(f32)
    r112 = lax.broadcasted_iota(jnp.int32, (112, 224), 0)
    c112 = lax.broadcasted_iota(jnp.int32, (112, 224), 1)
    Qe = (c112 == 2 * r112).astype(f32)
    Qo = (c112 == 2 * r112 + 1).astype(f32)
    xf = xb.reshape(96 * 2 * _R, 224)
    xje = _mm(xf, Pe).reshape(96, 2 * _R, 112)
    xjo = _mm(xf, Po).reshape(96, 2 * _R, 112)
    w112 = lax.broadcasted_iota(jnp.int32, (1, 112), 1).astype(f32)
    sn2 = jnp.sum(xsij_ref[0] * xsij_ref[0], axis=(0, 2)).reshape(_S, 1)
    iota_s = lax.broadcasted_iota(jnp.int32, (_S, 112), 0)
    zf = zft_ref[0]
    b2 = b2_ref[...]
    for hu in range(_R):
        innerT = jnp.zeros((_S, 112), f32)
        for i in range(2):
            rp = (i0 * 2 * _R + 2 * hu + i).astype(f32)
            for j in range(2):
                colpix = 2.0 * w112 + float(j)
                nrm = jnp.maximum(jnp.sqrt(rp * rp + colpix * colpix), 1e-12)
                xj = xje if j == 0 else xjo
                xij = jnp.concatenate(
                    [xj[:, 2 * hu + i, :], rp / nrm, colpix / nrm],
                    axis=0)
                innerT = innerT + _mm(xsij_ref[0, 2 * i + j], xij)
        neg = 2.0 * innerT - sn2
        accT = b2
        for k in range(_K):
            m = jnp.max(neg, axis=0, keepdims=True)
            hit = neg == m
            idx = jnp.min(jnp.where(hit, iota_s, _S), axis=0, keepdims=True)
            sel = iota_s == idx
            accT = accT + _mm(zf[k * _RO:(k + 1) * _RO], sel.astype(f32))
            neg = jnp.where(sel, -jnp.inf, neg)
        for i in range(2):
            row = (_mm(accT[(2 * i) * 96:(2 * i + 1) * 96], Qe)
                   + _mm(accT[(2 * i + 1) * 96:(2 * i + 2) * 96], Qo))
            out_ref[0, :, 2 * hu + i, :] = row


def kernel(x, Wc, bc, Wp, bp):
    B, Cin, H, W = x.shape
    Hu, Wu = H // 2, W // 2
    N = Hu * Wu
    f32 = jnp.float32
    ind = jnp.round(jnp.linspace(0, Hu - 1, 16)).astype(jnp.int32)
    xs4 = jnp.stack([x[:, :, 2 * ind + i, :][:, :, :, 2 * ind + j]
                     for i in range(2) for j in range(2)], axis=1)
    xg = jnp.arange(H, dtype=f32)
    coord_r = jnp.broadcast_to(xg[:, None], (H, W))
    coord_c = jnp.broadcast_to(xg[None, :], (H, W))
    nrm = jnp.maximum(jnp.sqrt(coord_r**2 + coord_c**2), 1e-12)
    cr, cc = coord_r / nrm, coord_c / nrm
    cs4 = jnp.stack([jnp.stack([cr[2 * ind + i, :][:, 2 * ind + j],
                                cc[2 * ind + i, :][:, 2 * ind + j]])
                     for i in range(2) for j in range(2)], axis=0)
    cs4 = jnp.broadcast_to(cs4[None], (B, 4, 2, 16, 16))
    xsij = jnp.concatenate([xs4, cs4], axis=2)
    xsij = xsij.reshape(B, 4, 98, _S)
    xsijT = xsij.transpose(0, 1, 3, 2)
    xs = xsij.transpose(0, 3, 2, 1).reshape(B, _S, 98 * 4)
    Wc4 = Wc.reshape(Cin + 2, 4, _C, _K)
    Wz = jnp.einsum('op,prck->ckro', Wp, Wc4).reshape(_C, _K * _RO)
    WzT = Wz.T
    b2 = (jnp.einsum('op,pr->ro', Wp, bc.reshape(Cin + 2, 4))
          + bp[None, :]).reshape(_RO, 1)
    xst = xs.transpose(0, 2, 1)

    zft = pl.pallas_call(
        _zft_body,
        grid=(B,),
        in_specs=[
            pl.BlockSpec((_K * _RO, _C), lambda b: (0, 0)),
            pl.BlockSpec((1, _C, _S), lambda b: (b, 0, 0)),
        ],
        out_specs=pl.BlockSpec((1, _K * _RO, _S), lambda b: (b, 0, 0)),
        out_shape=jax.ShapeDtypeStruct((B, _K * _RO, _S), f32),
    )(WzT, xst)

    out = pl.pallas_call(
        _fused_body,
        grid=(B, Hu // _R),
        in_specs=[
            pl.BlockSpec((1, Cin, 2 * _R, W), lambda b, i: (b, 0, i, 0)),
            pl.BlockSpec((1, 4, _S, 98), lambda b, i: (b, 0, 0, 0)),
            pl.BlockSpec((1, _K * _RO, _S), lambda b, i: (b, 0, 0)),
            pl.BlockSpec((_RO, 1), lambda b, i: (0, 0)),
        ],
        out_specs=pl.BlockSpec((1, 96, 2 * _R, W), lambda b, i: (b, 0, i, 0)),
        out_shape=jax.ShapeDtypeStruct((B, 96, H, W), f32),
    )(x, xsijT, zft, b2)

    return out
